# baseline (speedup 1.0000x reference)
import functools

import jax
import jax.numpy as jnp
from jax import lax
from jax.experimental import pallas as pl
from jax.experimental.pallas import tpu as pltpu
from jax.experimental.pallas import tpu_sc as plsc

N = 10000
E = 320000
FEATS = 128
HIDDEN = 64
CLASSES = 64
NTRAIN = 1000
DEPTH = 10

NCORE = 2
NSUB = 16
HALF = CLASSES // NCORE
ROWS_PT = 632
N_PAD = ROWS_PT * NSUB
EDGE_DUMP = N
TRAIN_DUMP = N + 100
CHUNK = 256
NBUF = 4
GDEPTH = 2
IDEPTH = 3
EDGES_PT = E // NSUB
NCH = 80
TRAIN_PT = 64


def _mlp_body(x_ref, w1_ref, b1_ref, w2_ref, b2_ref, p_ref):
    h = jnp.maximum(
        jnp.dot(x_ref[...], w1_ref[...], preferred_element_type=jnp.float32)
        + b1_ref[...], 0.0)
    lg = (jnp.dot(h, w2_ref[...], preferred_element_type=jnp.float32)
          + b2_ref[...])
    m = jnp.max(lg, axis=1, keepdims=True)
    e = jnp.exp(lg - m)
    p_ref[...] = e / jnp.sum(e, axis=1, keepdims=True)


def _mlp(x, W1, b1, W2, b2):
    blk = 1000
    return pl.pallas_call(
        _mlp_body,
        grid=(N // blk,),
        in_specs=[
            pl.BlockSpec((blk, FEATS), lambda i: (i, 0)),
            pl.BlockSpec((FEATS, HIDDEN), lambda i: (0, 0)),
            pl.BlockSpec((1, HIDDEN), lambda i: (0, 0)),
            pl.BlockSpec((HIDDEN, CLASSES), lambda i: (0, 0)),
            pl.BlockSpec((1, CLASSES), lambda i: (0, 0)),
        ],
        out_specs=pl.BlockSpec((blk, CLASSES), lambda i: (i, 0)),
        out_shape=jax.ShapeDtypeStruct((N, CLASSES), jnp.float32),
    )(x, W1, b1.reshape(1, HIDDEN), W2, b2.reshape(1, CLASSES))


def _log_body(s_ref, o_ref):
    blk = s_ref[...]
    o_ref[...] = jnp.log(jnp.concatenate([blk[0], blk[1]], axis=1) + 1.0)


def _log1p(state):
    blk = 1000
    return pl.pallas_call(
        _log_body,
        grid=(N // blk,),
        in_specs=[pl.BlockSpec((2, blk, HALF), lambda i: (0, i, 0))],
        out_specs=pl.BlockSpec((blk, CLASSES), lambda i: (i, 0)),
        out_shape=jax.ShapeDtypeStruct((N, CLASSES), jnp.float32),
    )(state)


_ZV = functools.partial(jnp.zeros, (16,), dtype=jnp.float32)


def _sc_body(p_st, srcr, dstr, tixr, labr, state,
             acc, six, dst_v, rows4, bufA, h0b_v, h0t_v,
             tix_v, lab_v, g_v, gsem, ssem, isem, sem):
    cid = lax.axis_index("c")
    sid = lax.axis_index("s")
    r0 = sid * ROWS_PT

    pltpu.sync_copy(dstr.at[sid], dst_v)
    pltpu.sync_copy(tixr.at[sid], tix_v)
    pltpu.sync_copy(labr.at[sid], lab_v)

    def _fill(i, _):
        rows4[0, i, pl.ds(0, 16)] = jnp.full((16,), 1.0, jnp.float32)
        rows4[0, i, pl.ds(16, 16)] = jnp.full((16,), 1.0, jnp.float32)
        return 0
    lax.fori_loop(0, CHUNK, _fill, 0)

    def _zero_bufA():
        def zb(i, _):
            bufA[i, pl.ds(0, 16)] = _ZV()
            bufA[i, pl.ds(16, 16)] = _ZV()
            return 0
        lax.fori_loop(0, ROWS_PT, zb, 0)

    _zero_bufA()
    pltpu.sync_copy(bufA, acc.at[pl.ds(r0, ROWS_PT)])
    plsc.subcore_barrier()

    def dstep(j, _):
        pltpu.sync_copy(rows4.at[0], acc.at[dst_v.at[j]], add=True)
        return 0
    lax.fori_loop(0, NCH, dstep, 0)
    plsc.subcore_barrier()

    pltpu.sync_copy(acc.at[pl.ds(r0, ROWS_PT)], bufA)

    def gstep(r, _):
        g_v[r, pl.ds(0, 16)] = 1.0 / jnp.maximum(bufA[r, pl.ds(0, 16)], 1.0)
        return 0
    lax.fori_loop(0, ROWS_PT, gstep, 0)
    _zero_bufA()
    pltpu.sync_copy(bufA, acc.at[pl.ds(r0, ROWS_PT)])

    pltpu.sync_copy(p_st.at[cid, pl.ds(r0, ROWS_PT)], bufA)

    def nstep(r, _):
        bufA[r, pl.ds(0, 16)] = -bufA[r, pl.ds(0, 16)]
        bufA[r, pl.ds(16, 16)] = -bufA[r, pl.ds(16, 16)]
        return 0
    lax.fori_loop(0, ROWS_PT, nstep, 0)
    pltpu.sync_copy(bufA, state.at[cid, pl.ds(r0, ROWS_PT)])

    pltpu.async_copy(p_st.at[cid].at[tix_v], h0t_v, sem).wait()
    iota = lax.iota(jnp.int32, 16)
    for i in range(TRAIN_PT):
        lv = lab_v[pl.ds((i // 16) * 16, 16)]
        lab = lv[i % 16] - HALF * cid
        v0 = -h0t_v[i, pl.ds(0, 16)]
        v1 = -h0t_v[i, pl.ds(16, 16)]
        h0t_v[i, pl.ds(0, 16)] = v0 + jnp.where(iota == lab, 1.0, 0.0)
        h0t_v[i, pl.ds(16, 16)] = v1 + jnp.where(iota + 16 == lab, 1.0, 0.0)
    plsc.subcore_barrier()
    pltpu.sync_copy(h0t_v, state.at[cid].at[tix_v])
    plsc.subcore_barrier()

    def _idx(t):
        pltpu.async_copy(srcr.at[sid, t], six.at[t % NBUF],
                         isem.at[t % NBUF])

    def _gather(t):
        b = t % NBUF
        pltpu.async_copy(state.at[cid].at[six.at[b]], rows4.at[b],
                         gsem.at[b])

    def _conv():
        for t in range(IDEPTH):
            _idx(t)
        for t in range(GDEPTH):
            pltpu.make_async_copy(srcr.at[sid, t], six.at[t], isem.at[t]
                                  ).wait()
            _gather(t)

        def group(jj, _):
            t0 = jj * NBUF
            for b in range(NBUF):
                t = t0 + b
                bg = (b + GDEPTH) % NBUF
                tn = t + GDEPTH

                pltpu.make_async_copy(
                    state.at[cid].at[six.at[b]], rows4.at[b], gsem.at[b]
                ).wait()
                pltpu.async_copy(rows4.at[b], acc.at[dst_v.at[t]],
                                 ssem.at[b], add=True)

                @pl.when(tn < NCH)
                def _():
                    pltpu.make_async_copy(
                        srcr.at[sid, tn], six.at[bg], isem.at[bg]
                    ).wait()

                    @pl.when(tn >= NBUF)
                    def _():
                        pltpu.make_async_copy(
                            rows4.at[bg], acc.at[dst_v.at[tn - NBUF]],
                            ssem.at[bg]
                        ).wait()
                    _gather(tn)

                @pl.when(t + IDEPTH < NCH)
                def _():
                    _idx(t + IDEPTH)
            return 0
        lax.fori_loop(0, NCH // NBUF, group, 0)
        for b in range(NBUF):
            t = NCH - NBUF + b
            pltpu.make_async_copy(
                rows4.at[t % NBUF], acc.at[dst_v.at[t]], ssem.at[t % NBUF]
            ).wait()

    def iter1(_, c):
        _conv()
        plsc.subcore_barrier()
        pltpu.sync_copy(acc.at[pl.ds(r0, ROWS_PT)], bufA)

        def fstep(r, _):
            g = g_v[r, pl.ds(0, 16)]
            bufA[r, pl.ds(0, 16)] = bufA[r, pl.ds(0, 16)] * g
            bufA[r, pl.ds(16, 16)] = bufA[r, pl.ds(16, 16)] * g
            return 0
        lax.fori_loop(0, ROWS_PT, fstep, 0)
        pltpu.sync_copy(bufA, state.at[cid, pl.ds(r0, ROWS_PT)])
        _zero_bufA()
        pltpu.sync_copy(bufA, acc.at[pl.ds(r0, ROWS_PT)])
        plsc.subcore_barrier()
        pltpu.sync_copy(h0t_v, state.at[cid].at[tix_v])
        plsc.subcore_barrier()
        return c
    lax.fori_loop(0, DEPTH, iter1, 0)

    pltpu.sync_copy(state.at[cid, pl.ds(r0, ROWS_PT)], h0b_v)
    pltpu.sync_copy(p_st.at[cid, pl.ds(r0, ROWS_PT)], bufA)

    def hstep(r, _):
        s0 = h0b_v[r, pl.ds(0, 16)] + bufA[r, pl.ds(0, 16)]
        s1 = h0b_v[r, pl.ds(16, 16)] + bufA[r, pl.ds(16, 16)]
        bufA[r, pl.ds(0, 16)] = s0
        bufA[r, pl.ds(16, 16)] = s1
        h0b_v[r, pl.ds(0, 16)] = s0 * 0.1
        h0b_v[r, pl.ds(16, 16)] = s1 * 0.1
        g_v[r, pl.ds(0, 16)] = g_v[r, pl.ds(0, 16)] * 0.9
        return 0
    lax.fori_loop(0, ROWS_PT, hstep, 0)
    pltpu.sync_copy(bufA, state.at[cid, pl.ds(r0, ROWS_PT)])
    _zero_bufA()
    plsc.subcore_barrier()

    def iter2(_, c):
        _conv()
        plsc.subcore_barrier()
        pltpu.sync_copy(acc.at[pl.ds(r0, ROWS_PT)], bufA)

        def fstep(r, _):
            g = g_v[r, pl.ds(0, 16)]
            bufA[r, pl.ds(0, 16)] = (bufA[r, pl.ds(0, 16)] * g
                                     + h0b_v[r, pl.ds(0, 16)])
            bufA[r, pl.ds(16, 16)] = (bufA[r, pl.ds(16, 16)] * g
                                      + h0b_v[r, pl.ds(16, 16)])
            return 0
        lax.fori_loop(0, ROWS_PT, fstep, 0)
        pltpu.sync_copy(bufA, state.at[cid, pl.ds(r0, ROWS_PT)])
        _zero_bufA()
        pltpu.sync_copy(bufA, acc.at[pl.ds(r0, ROWS_PT)])
        plsc.subcore_barrier()
        return c
    lax.fori_loop(0, DEPTH, iter2, 0)


def _sc_diffuse(p_st, srcr, dstr, tixr, labr):
    mesh = plsc.VectorSubcoreMesh(core_axis_name="c", subcore_axis_name="s")
    return pl.kernel(
        _sc_body,
        out_type=jax.ShapeDtypeStruct((NCORE, N_PAD, HALF), jnp.float32),
        mesh=mesh,
        scratch_types=[
            pltpu.VMEM_SHARED((N_PAD, HALF), jnp.float32),
            pltpu.VMEM((NBUF, CHUNK), jnp.int32),
            pltpu.VMEM((NCH, CHUNK), jnp.int32),
            pltpu.VMEM((NBUF, CHUNK, HALF), jnp.float32),
            pltpu.VMEM((ROWS_PT, HALF), jnp.float32),
            pltpu.VMEM((ROWS_PT, HALF), jnp.float32),
            pltpu.VMEM((TRAIN_PT, HALF), jnp.float32),
            pltpu.VMEM((TRAIN_PT,), jnp.int32),
            pltpu.VMEM((TRAIN_PT,), jnp.int32),
            pltpu.VMEM((ROWS_PT, 16), jnp.float32),
            pltpu.SemaphoreType.DMA((NBUF,)),
            pltpu.SemaphoreType.DMA((NBUF,)),
            pltpu.SemaphoreType.DMA((NBUF,)),
            pltpu.SemaphoreType.DMA,
        ],
        compiler_params=pltpu.CompilerParams(use_tc_tiling_on_sc=False),
    )(p_st, srcr, dstr, tixr, labr)


def kernel(x, edge_index, train_idx, labels, W1, b1, W2, b2):
    p = _mlp(x, W1, b1, W2, b2)
    p_pad = jnp.pad(p, ((0, N_PAD - N), (0, 0)))
    p_st = jnp.stack([p_pad[:, :HALF], p_pad[:, HALF:]])

    src = edge_index[0].astype(jnp.int32)
    dst = edge_index[1].astype(jnp.int32)
    order = jnp.argsort(src)
    src = src[order]
    dst = dst[order]
    pad_e = NCH * CHUNK - EDGES_PT
    srcr = jnp.pad(src.reshape(NSUB, EDGES_PT),
                   ((0, 0), (0, pad_e))).reshape(NSUB, NCH, CHUNK)
    dstr = jnp.pad(dst.reshape(NSUB, EDGES_PT), ((0, 0), (0, pad_e)),
                   constant_values=EDGE_DUMP).reshape(NSUB, NCH, CHUNK)

    pad_t = NSUB * TRAIN_PT - NTRAIN
    tixr = jnp.concatenate(
        [train_idx.astype(jnp.int32),
         jnp.full((pad_t,), TRAIN_DUMP, jnp.int32)]).reshape(NSUB, TRAIN_PT)
    labr = jnp.concatenate(
        [labels.astype(jnp.int32),
         jnp.zeros((pad_t,), jnp.int32)]).reshape(NSUB, TRAIN_PT)

    state = _sc_diffuse(p_st, srcr, dstr, tixr, labr)
    return _log1p(state[:, :N, :])


# speedup vs baseline: 1.4051x; 1.4051x over previous
import functools

import jax
import jax.numpy as jnp
from jax import lax
from jax.experimental import pallas as pl
from jax.experimental.pallas import tpu as pltpu
from jax.experimental.pallas import tpu_sc as plsc

N = 10000
E = 320000
FEATS = 128
HIDDEN = 64
CLASSES = 64
NTRAIN = 1000
DEPTH = 10

NCORE = 2
NSUB = 16
HALF = CLASSES // NCORE
ROWS_PT = 632
N_PAD = ROWS_PT * NSUB
EDGE_DUMP = N
TRAIN_DUMP = N + 100
CHUNK = 256
NBUF = 4
GDEPTH = 3
IDEPTH = 4
EDGES_PT = E // NSUB
NCH = 80
TRAIN_PT = 64


def _mlp_body(x_ref, w1_ref, b1_ref, w2_ref, b2_ref, p_ref):
    h = jnp.maximum(
        jnp.dot(x_ref[...], w1_ref[...], preferred_element_type=jnp.float32)
        + b1_ref[...], 0.0)
    lg = (jnp.dot(h, w2_ref[...], preferred_element_type=jnp.float32)
          + b2_ref[...])
    m = jnp.max(lg, axis=1, keepdims=True)
    e = jnp.exp(lg - m)
    p_ref[...] = e / jnp.sum(e, axis=1, keepdims=True)


def _mlp(x, W1, b1, W2, b2):
    blk = 1000
    return pl.pallas_call(
        _mlp_body,
        grid=(N // blk,),
        in_specs=[
            pl.BlockSpec((blk, FEATS), lambda i: (i, 0)),
            pl.BlockSpec((FEATS, HIDDEN), lambda i: (0, 0)),
            pl.BlockSpec((1, HIDDEN), lambda i: (0, 0)),
            pl.BlockSpec((HIDDEN, CLASSES), lambda i: (0, 0)),
            pl.BlockSpec((1, CLASSES), lambda i: (0, 0)),
        ],
        out_specs=pl.BlockSpec((blk, CLASSES), lambda i: (i, 0)),
        out_shape=jax.ShapeDtypeStruct((N, CLASSES), jnp.float32),
    )(x, W1, b1.reshape(1, HIDDEN), W2, b2.reshape(1, CLASSES))


def _log_body(s_ref, o_ref):
    blk = s_ref[...]
    o_ref[...] = jnp.log(jnp.concatenate([blk[0], blk[1]], axis=1) + 1.0)


def _log1p(state):
    blk = 1000
    return pl.pallas_call(
        _log_body,
        grid=(N // blk,),
        in_specs=[pl.BlockSpec((2, blk, HALF), lambda i: (0, i, 0))],
        out_specs=pl.BlockSpec((blk, CLASSES), lambda i: (i, 0)),
        out_shape=jax.ShapeDtypeStruct((N, CLASSES), jnp.float32),
    )(state)


_ZV = functools.partial(jnp.zeros, (16,), dtype=jnp.float32)


def _sc_body(p_st, srcr, dstr, tixr, labr, state,
             acc, six, dst_v, rows4, bufA, h0b_v, h0t_v,
             tix_v, lab_v, g_v, gsem, ssem, isem, sem):
    cid = lax.axis_index("c")
    sid = lax.axis_index("s")
    r0 = sid * ROWS_PT

    pltpu.sync_copy(dstr.at[sid], dst_v)
    pltpu.sync_copy(tixr.at[sid], tix_v)
    pltpu.sync_copy(labr.at[sid], lab_v)

    def _fill(i, _):
        rows4[0, i, pl.ds(0, 16)] = jnp.full((16,), 1.0, jnp.float32)
        rows4[0, i, pl.ds(16, 16)] = jnp.full((16,), 1.0, jnp.float32)
        return 0
    lax.fori_loop(0, CHUNK, _fill, 0)

    def _zero_bufA():
        def zb(i, _):
            bufA[i, pl.ds(0, 16)] = _ZV()
            bufA[i, pl.ds(16, 16)] = _ZV()
            return 0
        lax.fori_loop(0, ROWS_PT, zb, 0)

    _zero_bufA()
    pltpu.sync_copy(bufA, acc.at[pl.ds(r0, ROWS_PT)])
    plsc.subcore_barrier()

    def dstep(j, _):
        pltpu.sync_copy(rows4.at[0], acc.at[dst_v.at[j]], add=True)
        return 0
    lax.fori_loop(0, NCH, dstep, 0)
    plsc.subcore_barrier()

    pltpu.sync_copy(acc.at[pl.ds(r0, ROWS_PT)], bufA)

    def gstep(r, _):
        g_v[r, pl.ds(0, 16)] = 1.0 / jnp.maximum(bufA[r, pl.ds(0, 16)], 1.0)
        return 0
    lax.fori_loop(0, ROWS_PT, gstep, 0)
    _zero_bufA()
    pltpu.sync_copy(bufA, acc.at[pl.ds(r0, ROWS_PT)])

    pltpu.sync_copy(p_st.at[cid, pl.ds(r0, ROWS_PT)], bufA)

    def nstep(r, _):
        bufA[r, pl.ds(0, 16)] = -bufA[r, pl.ds(0, 16)]
        bufA[r, pl.ds(16, 16)] = -bufA[r, pl.ds(16, 16)]
        return 0
    lax.fori_loop(0, ROWS_PT, nstep, 0)
    pltpu.sync_copy(bufA, state.at[cid, pl.ds(r0, ROWS_PT)])

    pltpu.async_copy(p_st.at[cid].at[tix_v], h0t_v, sem).wait()
    iota = lax.iota(jnp.int32, 16)
    for i in range(TRAIN_PT):
        lv = lab_v[pl.ds((i // 16) * 16, 16)]
        lab = lv[i % 16] - HALF * cid
        v0 = -h0t_v[i, pl.ds(0, 16)]
        v1 = -h0t_v[i, pl.ds(16, 16)]
        h0t_v[i, pl.ds(0, 16)] = v0 + jnp.where(iota == lab, 1.0, 0.0)
        h0t_v[i, pl.ds(16, 16)] = v1 + jnp.where(iota + 16 == lab, 1.0, 0.0)
    plsc.subcore_barrier()
    pltpu.sync_copy(h0t_v, state.at[cid].at[tix_v])
    plsc.subcore_barrier()

    def _idx(t):
        pltpu.async_copy(srcr.at[sid, t], six.at[t % NBUF],
                         isem.at[t % NBUF])

    def _gather(t):
        b = t % NBUF
        pltpu.async_copy(state.at[cid].at[six.at[b]], rows4.at[b],
                         gsem.at[b])

    def _conv():
        for t in range(IDEPTH):
            _idx(t)
        for t in range(GDEPTH):
            pltpu.make_async_copy(srcr.at[sid, t], six.at[t], isem.at[t]
                                  ).wait()
            _gather(t)

        def group(jj, _):
            t0 = jj * NBUF
            for b in range(NBUF):
                t = t0 + b
                bg = (b + GDEPTH) % NBUF
                tn = t + GDEPTH

                pltpu.make_async_copy(
                    state.at[cid].at[six.at[b]], rows4.at[b], gsem.at[b]
                ).wait()
                pltpu.async_copy(rows4.at[b], acc.at[dst_v.at[t]],
                                 ssem.at[b], add=True)

                @pl.when(tn < NCH)
                def _():
                    pltpu.make_async_copy(
                        srcr.at[sid, tn], six.at[bg], isem.at[bg]
                    ).wait()

                    @pl.when(tn >= NBUF)
                    def _():
                        pltpu.make_async_copy(
                            rows4.at[bg], acc.at[dst_v.at[tn - NBUF]],
                            ssem.at[bg]
                        ).wait()
                    _gather(tn)

                @pl.when(t + IDEPTH < NCH)
                def _():
                    _idx(t + IDEPTH)
            return 0
        lax.fori_loop(0, NCH // NBUF, group, 0)
        for b in range(NBUF):
            t = NCH - NBUF + b
            pltpu.make_async_copy(
                rows4.at[t % NBUF], acc.at[dst_v.at[t]], ssem.at[t % NBUF]
            ).wait()

    def iter1(_, c):
        _conv()
        plsc.subcore_barrier()
        pltpu.sync_copy(acc.at[pl.ds(r0, ROWS_PT)], bufA)

        def fstep(r, _):
            g = g_v[r, pl.ds(0, 16)]
            bufA[r, pl.ds(0, 16)] = bufA[r, pl.ds(0, 16)] * g
            bufA[r, pl.ds(16, 16)] = bufA[r, pl.ds(16, 16)] * g
            return 0
        lax.fori_loop(0, ROWS_PT, fstep, 0)
        pltpu.sync_copy(bufA, state.at[cid, pl.ds(r0, ROWS_PT)])
        _zero_bufA()
        pltpu.sync_copy(bufA, acc.at[pl.ds(r0, ROWS_PT)])
        plsc.subcore_barrier()
        pltpu.sync_copy(h0t_v, state.at[cid].at[tix_v])
        plsc.subcore_barrier()
        return c
    lax.fori_loop(0, DEPTH, iter1, 0)

    pltpu.sync_copy(state.at[cid, pl.ds(r0, ROWS_PT)], h0b_v)
    pltpu.sync_copy(p_st.at[cid, pl.ds(r0, ROWS_PT)], bufA)

    def hstep(r, _):
        s0 = h0b_v[r, pl.ds(0, 16)] + bufA[r, pl.ds(0, 16)]
        s1 = h0b_v[r, pl.ds(16, 16)] + bufA[r, pl.ds(16, 16)]
        bufA[r, pl.ds(0, 16)] = s0
        bufA[r, pl.ds(16, 16)] = s1
        h0b_v[r, pl.ds(0, 16)] = s0 * 0.1
        h0b_v[r, pl.ds(16, 16)] = s1 * 0.1
        g_v[r, pl.ds(0, 16)] = g_v[r, pl.ds(0, 16)] * 0.9
        return 0
    lax.fori_loop(0, ROWS_PT, hstep, 0)
    pltpu.sync_copy(bufA, state.at[cid, pl.ds(r0, ROWS_PT)])
    _zero_bufA()
    plsc.subcore_barrier()

    def iter2(_, c):
        _conv()
        plsc.subcore_barrier()
        pltpu.sync_copy(acc.at[pl.ds(r0, ROWS_PT)], bufA)

        def fstep(r, _):
            g = g_v[r, pl.ds(0, 16)]
            bufA[r, pl.ds(0, 16)] = (bufA[r, pl.ds(0, 16)] * g
                                     + h0b_v[r, pl.ds(0, 16)])
            bufA[r, pl.ds(16, 16)] = (bufA[r, pl.ds(16, 16)] * g
                                      + h0b_v[r, pl.ds(16, 16)])
            return 0
        lax.fori_loop(0, ROWS_PT, fstep, 0)
        pltpu.sync_copy(bufA, state.at[cid, pl.ds(r0, ROWS_PT)])
        _zero_bufA()
        pltpu.sync_copy(bufA, acc.at[pl.ds(r0, ROWS_PT)])
        plsc.subcore_barrier()
        return c
    lax.fori_loop(0, DEPTH, iter2, 0)


def _sc_diffuse(p_st, srcr, dstr, tixr, labr):
    mesh = plsc.VectorSubcoreMesh(core_axis_name="c", subcore_axis_name="s")
    return pl.kernel(
        _sc_body,
        out_type=jax.ShapeDtypeStruct((NCORE, N_PAD, HALF), jnp.float32),
        mesh=mesh,
        scratch_types=[
            pltpu.VMEM_SHARED((N_PAD, HALF), jnp.float32),
            pltpu.VMEM((NBUF, CHUNK), jnp.int32),
            pltpu.VMEM((NCH, CHUNK), jnp.int32),
            pltpu.VMEM((NBUF, CHUNK, HALF), jnp.float32),
            pltpu.VMEM((ROWS_PT, HALF), jnp.float32),
            pltpu.VMEM((ROWS_PT, HALF), jnp.float32),
            pltpu.VMEM((TRAIN_PT, HALF), jnp.float32),
            pltpu.VMEM((TRAIN_PT,), jnp.int32),
            pltpu.VMEM((TRAIN_PT,), jnp.int32),
            pltpu.VMEM((ROWS_PT, 16), jnp.float32),
            pltpu.SemaphoreType.DMA((NBUF,)),
            pltpu.SemaphoreType.DMA((NBUF,)),
            pltpu.SemaphoreType.DMA((NBUF,)),
            pltpu.SemaphoreType.DMA,
        ],
        compiler_params=pltpu.CompilerParams(use_tc_tiling_on_sc=False),
    )(p_st, srcr, dstr, tixr, labr)


def kernel(x, edge_index, train_idx, labels, W1, b1, W2, b2):
    p = _mlp(x, W1, b1, W2, b2)
    p_pad = jnp.pad(p, ((0, N_PAD - N), (0, 0)))
    p_st = jnp.stack([p_pad[:, :HALF], p_pad[:, HALF:]])

    src = edge_index[0].astype(jnp.int32)
    dst = edge_index[1].astype(jnp.int32)
    order = jnp.argsort(dst)
    src = src[order]
    dst = dst[order]
    pad_e = NCH * CHUNK - EDGES_PT
    srcr = jnp.pad(src.reshape(NSUB, EDGES_PT),
                   ((0, 0), (0, pad_e))).reshape(NSUB, NCH, CHUNK)
    dstr = jnp.pad(dst.reshape(NSUB, EDGES_PT), ((0, 0), (0, pad_e)),
                   constant_values=EDGE_DUMP).reshape(NSUB, NCH, CHUNK)

    pad_t = NSUB * TRAIN_PT - NTRAIN
    tixr = jnp.concatenate(
        [train_idx.astype(jnp.int32),
         jnp.full((pad_t,), TRAIN_DUMP, jnp.int32)]).reshape(NSUB, TRAIN_PT)
    labr = jnp.concatenate(
        [labels.astype(jnp.int32),
         jnp.zeros((pad_t,), jnp.int32)]).reshape(NSUB, TRAIN_PT)

    state = _sc_diffuse(p_st, srcr, dstr, tixr, labr)
    return _log1p(state[:, :N, :])
